# baseline (speedup 1.0000x reference)
import functools

import jax
import jax.numpy as jnp
from jax import lax
from jax.experimental import pallas as pl
from jax.experimental.pallas import tpu as pltpu
from jax.experimental.pallas import tpu_sc as plsc

VOCAB = 400004
EMB = 64
START_IDX = 400001
END_IDX = 400002

ROWS = 4096
COLS = 200
B_TOT = ROWS * COLS
NC = 2
NS = 16
NW = NC * NS
B_W = B_TOT // NW
CH = 128
NCH = B_W // CH


def _sc_body(idx_hbm, table_hbm, marker_hbm, out_hbm, idx_v, rv, mk_v, gsem):
    wid = lax.axis_index("s") * NC + lax.axis_index("c")
    row0 = wid * NCH
    base = wid * B_W

    pltpu.sync_copy(idx_hbm.at[pl.ds(row0, NCH)], idx_v)
    pltpu.sync_copy(marker_hbm, mk_v)
    mk = [[mk_v[r, pl.ds(cc * 16, 16)] for cc in range(4)] for r in range(2)]

    def chunk(c, carry):
        pltpu.async_copy(table_hbm.at[idx_v.at[c]], rv, gsem).wait()
        for g in range(8):
            v = idx_v[c, pl.ds(g * 16, 16)]
            m = (v == START_IDX) | (v == END_IDX)
            s = jnp.sum(m.astype(jnp.int32))

            @pl.when(s > 0)
            def _():
                for l in range(16):
                    sv = v[l]

                    @pl.when(sv == START_IDX)
                    def _():
                        for cc in range(4):
                            rv[g * 16 + l, pl.ds(cc * 16, 16)] = mk[0][cc]

                    @pl.when(sv == END_IDX)
                    def _():
                        for cc in range(4):
                            rv[g * 16 + l, pl.ds(cc * 16, 16)] = mk[1][cc]

        pltpu.sync_copy(rv, out_hbm.at[pl.ds(base + c * CH, CH)])
        return carry

    lax.fori_loop(0, NCH, chunk, 0)


@functools.partial(
    pl.kernel,
    mesh=plsc.VectorSubcoreMesh(core_axis_name="c", subcore_axis_name="s"),
    out_type=jax.ShapeDtypeStruct((B_TOT, EMB), jnp.float32),
    compiler_params=pltpu.CompilerParams(
        use_tc_tiling_on_sc=False, needs_layout_passes=False
    ),
    scratch_types=[
        pltpu.VMEM((NCH, CH), jnp.int32),
        pltpu.VMEM((CH, EMB), jnp.float32),
        pltpu.VMEM((2, EMB), jnp.float32),
        pltpu.SemaphoreType.DMA,
    ],
)
def _sc_gather(idx_hbm, table_hbm, marker_hbm, out_hbm, idx_v, rv, mk_v, gsem):
    _sc_body(idx_hbm, table_hbm, marker_hbm, out_hbm, idx_v, rv, mk_v, gsem)


def kernel(idxes, embeddings_weight, marker_weight):
    idx2d = idxes.reshape(B_TOT // CH, CH)
    out = _sc_gather(idx2d, embeddings_weight, marker_weight)
    return out.reshape(ROWS, COLS, EMB)


# speedup vs baseline: 1.0850x; 1.0850x over previous
import functools

import jax
import jax.numpy as jnp
from jax import lax
from jax.experimental import pallas as pl
from jax.experimental.pallas import tpu as pltpu
from jax.experimental.pallas import tpu_sc as plsc

VOCAB = 400004
EMB = 64
START_IDX = 400001
END_IDX = 400002

ROWS = 4096
COLS = 200
B_TOT = ROWS * COLS
NC = 2
NS = 16
NW = NC * NS
B_W = B_TOT // NW
CH = 128
NCH = B_W // CH
NBUF = 4
GAHEAD = 2


def _sc_body(idx_hbm, table_hbm, marker_hbm, out_hbm, idx_v, rvs, mk_v, gsems, ssems):
    wid = lax.axis_index("s") * NC + lax.axis_index("c")
    row0 = wid * NCH
    base = wid * B_W

    pltpu.sync_copy(idx_hbm.at[pl.ds(row0, NCH)], idx_v)
    pltpu.sync_copy(marker_hbm, mk_v)
    mk = [[mk_v[r, pl.ds(cc * 16, 16)] for cc in range(4)] for r in range(2)]

    def start_gather(c, b):
        pltpu.async_copy(table_hbm.at[idx_v.at[c]], rvs[b], gsems[b])

    def wait_gather(c, b):
        pltpu.make_async_copy(table_hbm.at[idx_v.at[c]], rvs[b], gsems[b]).wait()

    def start_scatter(c, b):
        pltpu.async_copy(rvs[b], out_hbm.at[pl.ds(base + c * CH, CH)], ssems[b])

    def wait_scatter(c, b):
        pltpu.make_async_copy(
            rvs[b], out_hbm.at[pl.ds(base + c * CH, CH)], ssems[b]
        ).wait()

    def fixup(c, b):
        rv = rvs[b]
        for g in range(8):
            v = idx_v[c, pl.ds(g * 16, 16)]
            m = (v == START_IDX) | (v == END_IDX)
            s = jnp.sum(m.astype(jnp.int32))

            @pl.when(s > 0)
            def _():
                for l in range(16):
                    sv = v[l]

                    @pl.when(sv == START_IDX)
                    def _():
                        for cc in range(4):
                            rv[g * 16 + l, pl.ds(cc * 16, 16)] = mk[0][cc]

                    @pl.when(sv == END_IDX)
                    def _():
                        for cc in range(4):
                            rv[g * 16 + l, pl.ds(cc * 16, 16)] = mk[1][cc]

    for b in range(GAHEAD):
        start_gather(b, b)

    def outer(o, carry):
        c0 = o * NBUF
        for db in range(NBUF):
            c = c0 + db
            wait_gather(c, db)
            fixup(c, db)
            start_scatter(c, db)
            nc = c + GAHEAD
            bb = (db + GAHEAD) % NBUF

            @pl.when(nc < NCH)
            def _():
                @pl.when(c >= NBUF - GAHEAD)
                def _():
                    wait_scatter(c - (NBUF - GAHEAD), bb)

                start_gather(nc, bb)

        return carry

    lax.fori_loop(0, NCH // NBUF, outer, 0)
    for b in range(NBUF):
        wait_scatter(NCH - NBUF + b, b)


@functools.partial(
    pl.kernel,
    mesh=plsc.VectorSubcoreMesh(core_axis_name="c", subcore_axis_name="s"),
    out_type=jax.ShapeDtypeStruct((B_TOT, EMB), jnp.float32),
    compiler_params=pltpu.CompilerParams(
        use_tc_tiling_on_sc=False, needs_layout_passes=False
    ),
    scratch_types=[
        pltpu.VMEM((NCH, CH), jnp.int32),
        pltpu.VMEM((CH, EMB), jnp.float32),
        pltpu.VMEM((CH, EMB), jnp.float32),
        pltpu.VMEM((CH, EMB), jnp.float32),
        pltpu.VMEM((CH, EMB), jnp.float32),
        pltpu.VMEM((2, EMB), jnp.float32),
        pltpu.SemaphoreType.DMA,
        pltpu.SemaphoreType.DMA,
        pltpu.SemaphoreType.DMA,
        pltpu.SemaphoreType.DMA,
        pltpu.SemaphoreType.DMA,
        pltpu.SemaphoreType.DMA,
        pltpu.SemaphoreType.DMA,
        pltpu.SemaphoreType.DMA,
    ],
)
def _sc_gather(
    idx_hbm, table_hbm, marker_hbm, out_hbm,
    idx_v, rv0, rv1, rv2, rv3, mk_v,
    g0, g1, g2, g3, s0, s1, s2, s3,
):
    _sc_body(
        idx_hbm, table_hbm, marker_hbm, out_hbm,
        idx_v, [rv0, rv1, rv2, rv3], mk_v,
        [g0, g1, g2, g3], [s0, s1, s2, s3],
    )


def kernel(idxes, embeddings_weight, marker_weight):
    idx2d = idxes.reshape(B_TOT // CH, CH)
    out = _sc_gather(idx2d, embeddings_weight, marker_weight)
    return out.reshape(ROWS, COLS, EMB)


# speedup vs baseline: 1.5735x; 1.4502x over previous
import functools

import jax
import jax.numpy as jnp
from jax import lax
from jax.experimental import pallas as pl
from jax.experimental.pallas import tpu as pltpu
from jax.experimental.pallas import tpu_sc as plsc

VOCAB = 400004
EMB = 64
START_IDX = 400001
END_IDX = 400002

ROWS = 4096
COLS = 200
B_TOT = ROWS * COLS
NC = 2
NS = 16
NW = NC * NS
B_W = B_TOT // NW
CH = 128
NCH = B_W // CH
NBUF = 4
GAHEAD = 2


def _sc_body(idx_hbm, table_hbm, marker_hbm, out_hbm, idx_v, rvs, mk_v, gsems, ssems):
    wid = lax.axis_index("s") * NC + lax.axis_index("c")
    row0 = wid * NCH
    base = wid * B_W

    pltpu.sync_copy(idx_hbm.at[pl.ds(row0, NCH)], idx_v)
    pltpu.sync_copy(marker_hbm, mk_v)
    mk = [[mk_v[r, pl.ds(cc * 16, 16)] for cc in range(4)] for r in range(2)]

    def start_gather(c, b):
        pltpu.async_copy(table_hbm.at[idx_v.at[c]], rvs[b], gsems[b])

    def wait_gather(c, b):
        pltpu.make_async_copy(table_hbm.at[idx_v.at[c]], rvs[b], gsems[b]).wait()

    def start_scatter(c, b):
        pltpu.async_copy(
            rvs[b], out_hbm.at[pl.ds(base + c * CH, CH), pl.ds(0, EMB)], ssems[b]
        )

    def wait_scatter(c, b):
        pltpu.make_async_copy(
            rvs[b], out_hbm.at[pl.ds(base + c * CH, CH), pl.ds(0, EMB)], ssems[b]
        ).wait()

    def fixup(c, b):
        rv = rvs[b]
        for g in range(8):
            v = idx_v[c, pl.ds(g * 16, 16)]
            m = (v == START_IDX) | (v == END_IDX)
            s = jnp.sum(m.astype(jnp.int32))

            @pl.when(s > 0)
            def _():
                for l in range(16):
                    sv = v[l]

                    @pl.when(sv == START_IDX)
                    def _():
                        for cc in range(4):
                            rv[g * 16 + l, pl.ds(cc * 16, 16)] = mk[0][cc]

                    @pl.when(sv == END_IDX)
                    def _():
                        for cc in range(4):
                            rv[g * 16 + l, pl.ds(cc * 16, 16)] = mk[1][cc]

    for b in range(GAHEAD):
        start_gather(b, b)

    def outer(o, carry):
        c0 = o * NBUF
        for db in range(NBUF):
            c = c0 + db
            wait_gather(c, db)
            fixup(c, db)
            start_scatter(c, db)
            nc = c + GAHEAD
            bb = (db + GAHEAD) % NBUF

            @pl.when(nc < NCH)
            def _():
                @pl.when(c >= NBUF - GAHEAD)
                def _():
                    wait_scatter(c - (NBUF - GAHEAD), bb)

                start_gather(nc, bb)

        return carry

    lax.fori_loop(0, NCH // NBUF, outer, 0)
    for b in range(NBUF):
        wait_scatter(NCH - NBUF + b, b)


@functools.partial(
    pl.kernel,
    mesh=plsc.VectorSubcoreMesh(core_axis_name="c", subcore_axis_name="s"),
    out_type=jax.ShapeDtypeStruct((B_TOT, 2 * EMB), jnp.float32),
    compiler_params=pltpu.CompilerParams(
        use_tc_tiling_on_sc=False, needs_layout_passes=False
    ),
    scratch_types=[
        pltpu.VMEM((NCH, CH), jnp.int32),
        pltpu.VMEM((CH, EMB), jnp.float32),
        pltpu.VMEM((CH, EMB), jnp.float32),
        pltpu.VMEM((CH, EMB), jnp.float32),
        pltpu.VMEM((CH, EMB), jnp.float32),
        pltpu.VMEM((2, EMB), jnp.float32),
        pltpu.SemaphoreType.DMA,
        pltpu.SemaphoreType.DMA,
        pltpu.SemaphoreType.DMA,
        pltpu.SemaphoreType.DMA,
        pltpu.SemaphoreType.DMA,
        pltpu.SemaphoreType.DMA,
        pltpu.SemaphoreType.DMA,
        pltpu.SemaphoreType.DMA,
    ],
)
def _sc_gather(
    idx_hbm, table_hbm, marker_hbm, out_hbm,
    idx_v, rv0, rv1, rv2, rv3, mk_v,
    g0, g1, g2, g3, s0, s1, s2, s3,
):
    _sc_body(
        idx_hbm, table_hbm, marker_hbm, out_hbm,
        idx_v, [rv0, rv1, rv2, rv3], mk_v,
        [g0, g1, g2, g3], [s0, s1, s2, s3],
    )


def kernel(idxes, embeddings_weight, marker_weight):
    idx2d = idxes.reshape(B_TOT // CH, CH)
    out = _sc_gather(idx2d, embeddings_weight, marker_weight)
    return out[:, :EMB].reshape(ROWS, COLS, EMB)


# speedup vs baseline: 1.8939x; 1.2037x over previous
import functools

import jax
import jax.numpy as jnp
from jax import lax
from jax.experimental import pallas as pl
from jax.experimental.pallas import tpu as pltpu
from jax.experimental.pallas import tpu_sc as plsc

VOCAB = 400004
EMB = 64
START_IDX = 400001
END_IDX = 400002

ROWS = 4096
COLS = 200
B_TOT = ROWS * COLS
NC = 2
NS = 16
NW = NC * NS
B_W = B_TOT // NW
CH = 256
NCH = B_W // CH
NBUF = 4
GAHEAD = 2
OUT_W = 2 * EMB


def _sc_body(idx_hbm, table_hbm, marker_hbm, out_hbm, idx_v, rvs, mk_v, gsems, ssems):
    wid = lax.axis_index("s") * NC + lax.axis_index("c")
    row0 = wid * NCH
    base = wid * B_W

    pltpu.sync_copy(idx_hbm.at[pl.ds(row0, NCH)], idx_v)
    pltpu.sync_copy(marker_hbm, mk_v)
    mk = [[mk_v[r, pl.ds(cc * 16, 16)] for cc in range(4)] for r in range(2)]

    def start_gather(c, b):
        pltpu.async_copy(
            table_hbm.at[idx_v.at[c, pl.ds(0, 128)]], rvs[b].at[pl.ds(0, 128)],
            gsems[b],
        )
        pltpu.async_copy(
            table_hbm.at[idx_v.at[c, pl.ds(128, 128)]], rvs[b].at[pl.ds(128, 128)],
            gsems[b],
        )

    def wait_gather(c, b):
        pltpu.make_async_copy(
            table_hbm.at[idx_v.at[c, pl.ds(0, 128)]], rvs[b].at[pl.ds(0, 128)],
            gsems[b],
        ).wait()
        pltpu.make_async_copy(
            table_hbm.at[idx_v.at[c, pl.ds(128, 128)]], rvs[b].at[pl.ds(128, 128)],
            gsems[b],
        ).wait()

    def start_scatter(c, b):
        pltpu.async_copy(
            rvs[b], out_hbm.at[pl.ds(base + c * CH, CH), pl.ds(0, EMB)], ssems[b]
        )

    def wait_scatter(c, b):
        pltpu.make_async_copy(
            rvs[b], out_hbm.at[pl.ds(base + c * CH, CH), pl.ds(0, EMB)], ssems[b]
        ).wait()

    def fixup(c, b):
        rv = rvs[b]
        vmax = idx_v[c, pl.ds(0, 16)]
        for g in range(1, 16):
            vmax = jnp.maximum(vmax, idx_v[c, pl.ds(g * 16, 16)])
        top = jnp.max(vmax)

        @pl.when(top >= START_IDX)
        def _():
            def group(g, carry):
                v = idx_v[c, pl.ds(g * 16, 16)]
                hits = jnp.max(v)

                @pl.when(hits >= START_IDX)
                def _():
                    for l in range(16):
                        row = g * 16 + l
                        sv = v[l]

                        @pl.when(sv == START_IDX)
                        def _():
                            for cc in range(4):
                                rv[row, pl.ds(cc * 16, 16)] = mk[0][cc]

                        @pl.when(sv == END_IDX)
                        def _():
                            for cc in range(4):
                                rv[row, pl.ds(cc * 16, 16)] = mk[1][cc]

                return carry

            lax.fori_loop(0, 16, group, 0)

    for b in range(GAHEAD):
        start_gather(b, b)

    def outer(o, carry):
        c0 = o * NBUF
        for db in range(NBUF):
            c = c0 + db
            wait_gather(c, db)
            fixup(c, db)
            start_scatter(c, db)
            nc = c + GAHEAD
            bb = (db + GAHEAD) % NBUF

            @pl.when(nc < NCH)
            def _():
                @pl.when(c >= NBUF - GAHEAD)
                def _():
                    wait_scatter(c - (NBUF - GAHEAD), bb)

                start_gather(nc, bb)

        return carry

    lax.fori_loop(0, NCH // NBUF, outer, 0)
    for b in range(NBUF):
        wait_scatter(NCH - NBUF + b, b)


@functools.partial(
    pl.kernel,
    mesh=plsc.VectorSubcoreMesh(core_axis_name="c", subcore_axis_name="s"),
    out_type=jax.ShapeDtypeStruct((B_TOT, OUT_W), jnp.float32),
    compiler_params=pltpu.CompilerParams(
        use_tc_tiling_on_sc=False, needs_layout_passes=False
    ),
    scratch_types=[
        pltpu.VMEM((NCH, CH), jnp.int32),
        pltpu.VMEM((CH, EMB), jnp.float32),
        pltpu.VMEM((CH, EMB), jnp.float32),
        pltpu.VMEM((CH, EMB), jnp.float32),
        pltpu.VMEM((CH, EMB), jnp.float32),
        pltpu.VMEM((2, EMB), jnp.float32),
        pltpu.SemaphoreType.DMA,
        pltpu.SemaphoreType.DMA,
        pltpu.SemaphoreType.DMA,
        pltpu.SemaphoreType.DMA,
        pltpu.SemaphoreType.DMA,
        pltpu.SemaphoreType.DMA,
        pltpu.SemaphoreType.DMA,
        pltpu.SemaphoreType.DMA,
    ],
)
def _sc_gather(
    idx_hbm, table_hbm, marker_hbm, out_hbm,
    idx_v, rv0, rv1, rv2, rv3, mk_v,
    g0, g1, g2, g3, s0, s1, s2, s3,
):
    _sc_body(
        idx_hbm, table_hbm, marker_hbm, out_hbm,
        idx_v, [rv0, rv1, rv2, rv3], mk_v,
        [g0, g1, g2, g3], [s0, s1, s2, s3],
    )


def kernel(idxes, embeddings_weight, marker_weight):
    idx2d = idxes.reshape(B_TOT // CH, CH)
    out = _sc_gather(idx2d, embeddings_weight, marker_weight)
    return out[:, :EMB].reshape(ROWS, COLS, EMB)
